# baseline (speedup 1.0000x reference)
import functools

import jax
import jax.numpy as jnp
from jax import lax
from jax.experimental import pallas as pl
from jax.experimental.pallas import tpu as pltpu
from jax.experimental.pallas import tpu_sc as plsc

N_NODES = 10000
D = 128
N_EDGES = 320000

NC = 2
NS = 16
NW = NC * NS
CHUNK = 128
NROWS = N_EDGES // CHUNK
KCH = 80
LAST_KCH = NROWS - (NW - 1) * KCH
NROWS_PAD = NW * KCH
NP = 10240
RPT = NP // NS


def _make_sc_aggregate(with_cnt):
    mesh = plsc.VectorSubcoreMesh(core_axis_name="c", subcore_axis_name="s")

    out_type = [jax.ShapeDtypeStruct((NC, NP, D), jnp.float32)]
    scratch = [
        pltpu.VMEM((KCH // 2, CHUNK), jnp.int32),
        pltpu.VMEM((KCH // 2, CHUNK), jnp.int32),
        pltpu.VMEM((CHUNK, D), jnp.float32),
        pltpu.VMEM((CHUNK, D), jnp.float32),
        pltpu.VMEM_SHARED((NP, D), jnp.float32),
        pltpu.SemaphoreType.DMA,
        pltpu.SemaphoreType.DMA,
    ]
    if with_cnt:
        out_type.append(jax.ShapeDtypeStruct((NC, NP), jnp.float32))
        scratch += [
            pltpu.VMEM((CHUNK,), jnp.float32),
            pltpu.VMEM((RPT,), jnp.float32),
            pltpu.VMEM_SHARED((NP,), jnp.float32),
        ]

    @functools.partial(pl.kernel, out_type=out_type, mesh=mesh,
                       scratch_types=scratch)
    def agg_kernel(x_hbm, e_hbm, out_agg, *rest):
        if with_cnt:
            out_cnt, si, di, rows_a, rows_b, acc, sem_a, sem_b, \
                ones_v, zv, cacc = rest
        else:
            si, di, rows_a, rows_b, acc, sem_a, sem_b = rest
        cid = lax.axis_index("c")
        sid = lax.axis_index("s")
        wid = sid * NC + cid
        nch = jnp.where(wid == NW - 1, LAST_KCH, KCH)
        tb = sid * RPT

        zero16 = jnp.zeros((16,), jnp.float32)

        def fill_rows(i, _):
            rows_a[i // 8, pl.ds((i % 8) * 16, 16)] = zero16
            return _
        lax.fori_loop(0, CHUNK * 8, fill_rows, None)

        if with_cnt:
            def fill_ones(i, _):
                ones_v[pl.ds(i * 16, 16)] = jnp.full((16,), 1.0, jnp.float32)
                return _
            lax.fori_loop(0, CHUNK // 16, fill_ones, None)

            def fill_z(i, _):
                zv[pl.ds(i * 16, 16)] = zero16
                return _
            lax.fori_loop(0, RPT // 16, fill_z, None)

        zcp = [pltpu.async_copy(rows_a, acc.at[pl.ds(tb + b * CHUNK, CHUNK)],
                                sem_a) for b in range(RPT // CHUNK)]
        if with_cnt:
            zcp.append(pltpu.async_copy(zv, cacc.at[pl.ds(tb, RPT)], sem_a))
        for c in zcp:
            c.wait()
        plsc.subcore_barrier()

        half = KCH // 2
        last = half - 1

        def half_body(h, _):
            base = wid * KCH + h * half
            s1 = pltpu.async_copy(e_hbm.at[0, pl.ds(base, half)], si, sem_a)
            s2 = pltpu.async_copy(e_hbm.at[1, pl.ds(base, half)], di, sem_b)
            s1.wait()
            s2.wait()
            pltpu.async_copy(x_hbm.at[si.at[0]], rows_a, sem_a).wait()
            npairs = jnp.clip(nch - h * half, 0, half) // 2

            def chunk_body(i, __):
                j = 2 * i
                nxt_b = pltpu.async_copy(x_hbm.at[si.at[j + 1]], rows_b, sem_b)
                pltpu.sync_copy(rows_a, acc.at[di.at[j]], add=True)
                if with_cnt:
                    pltpu.sync_copy(ones_v, cacc.at[di.at[j]], add=True)
                nxt_b.wait()
                nxt_a = pltpu.async_copy(
                    x_hbm.at[si.at[jnp.minimum(j + 2, last)]], rows_a, sem_a)
                pltpu.sync_copy(rows_b, acc.at[di.at[j + 1]], add=True)
                if with_cnt:
                    pltpu.sync_copy(ones_v, cacc.at[di.at[j + 1]], add=True)
                nxt_a.wait()
                return __
            lax.fori_loop(0, npairs, chunk_body, None)
            return _
        lax.fori_loop(0, 2, half_body, None)

        plsc.subcore_barrier()

        for b in range(RPT // CHUNK):
            pltpu.sync_copy(acc.at[pl.ds(tb + b * CHUNK, CHUNK)],
                            out_agg.at[cid, pl.ds(tb + b * CHUNK, CHUNK)])
        if with_cnt:
            pltpu.sync_copy(cacc.at[pl.ds(tb, RPT)],
                            out_cnt.at[cid, pl.ds(tb, RPT)])

    return agg_kernel


_sc_agg_cnt = _make_sc_aggregate(True)


_BLK = 1000
_GRID = N_NODES // _BLK


def _dot_t(a, w):
    return lax.dot_general(a, w, (((1,), (1,)), ((), ())),
                           preferred_element_type=jnp.float32)


def _root1_body(x_ref, wr_ref, b_ref, o_ref):
    o_ref[...] = _dot_t(x_ref[...], wr_ref[...]) + b_ref[...]


def _root2_body(h_ref, wr2_ref, b2_ref, wr3_ref, b3_ref, o2_ref, o3_ref):
    h = h_ref[...]
    o2_ref[...] = _dot_t(h, wr2_ref[...]) + b2_ref[...]
    o3_ref[...] = _dot_t(h, wr3_ref[...]) + b3_ref[...]


def _comb1_body(a_ref, c_ref, r_ref, wl_ref, o_ref):
    a = a_ref[0] + a_ref[1]
    c = c_ref[0] + c_ref[1]
    mean = a / jnp.maximum(c, 1.0)
    o_ref[...] = jnp.maximum(_dot_t(mean, wl_ref[...]) + r_ref[...], 0.0)


def _comb2_body(a_ref, c_ref, r2_ref, r3_ref, wl2_ref, wl3_ref,
                mu_ref, var_ref):
    a = a_ref[0] + a_ref[1]
    c = c_ref[0] + c_ref[1]
    mean = a / jnp.maximum(c, 1.0)
    mu_ref[...] = _dot_t(mean, wl2_ref[...]) + r2_ref[...]
    var_ref[...] = _dot_t(mean, wl3_ref[...]) + r3_ref[...]


def _agg_spec():
    return pl.BlockSpec((NC, _BLK, D), lambda i: (0, i, 0))


def _cnt_spec():
    return pl.BlockSpec((NC, _BLK, 1), lambda i: (0, i, 0))


def _row_spec():
    return pl.BlockSpec((_BLK, D), lambda i: (i, 0))


def _w_spec():
    return pl.BlockSpec((D, D), lambda i: (0, 0))


def _b_spec():
    return pl.BlockSpec((1, D), lambda i: (0, 0))


_ROW_SDS = jax.ShapeDtypeStruct((N_NODES, D), jnp.float32)


def _root1(x, Wr, bl):
    return pl.pallas_call(
        _root1_body,
        grid=(_GRID,),
        in_specs=[_row_spec(), _w_spec(), _b_spec()],
        out_specs=_row_spec(),
        out_shape=_ROW_SDS,
    )(x, Wr, bl.reshape(1, D))


def _root2(h, Wr2, bl2, Wr3, bl3):
    return pl.pallas_call(
        _root2_body,
        grid=(_GRID,),
        in_specs=[_row_spec(), _w_spec(), _b_spec(), _w_spec(), _b_spec()],
        out_specs=[_row_spec(), _row_spec()],
        out_shape=[_ROW_SDS, _ROW_SDS],
    )(h, Wr2, bl2.reshape(1, D), Wr3, bl3.reshape(1, D))


def _comb1(agg, cnt3, r1, Wl):
    return pl.pallas_call(
        _comb1_body,
        grid=(_GRID,),
        in_specs=[_agg_spec(), _cnt_spec(), _row_spec(), _w_spec()],
        out_specs=_row_spec(),
        out_shape=_ROW_SDS,
    )(agg, cnt3, r1, Wl)


def _comb2(agg, cnt3, r2, r3, Wl2, Wl3):
    return pl.pallas_call(
        _comb2_body,
        grid=(_GRID,),
        in_specs=[_agg_spec(), _cnt_spec(), _row_spec(), _row_spec(),
                  _w_spec(), _w_spec()],
        out_specs=[_row_spec(), _row_spec()],
        out_shape=[_ROW_SDS, _ROW_SDS],
    )(agg, cnt3, r2, r3, Wl2, Wl3)


def kernel(x, edge_index, edge_weight, Wl1, bl1, Wr1, Wl2, bl2, Wr2,
           Wl3, bl3, Wr3):
    e3 = jnp.pad(edge_index.astype(jnp.int32).reshape(2, NROWS, CHUNK),
                 ((0, 0), (0, NROWS_PAD - NROWS), (0, 0)))

    r1 = _root1(x, Wr1, bl1)
    agg1, cnt = _sc_agg_cnt(x, e3)
    cnt3 = cnt[:, :, None]
    h1 = _comb1(agg1, cnt3, r1, Wl1)
    r2, r3 = _root2(h1, Wr2, bl2, Wr3, bl3)
    agg2, _ = _sc_agg_cnt(h1, e3)
    mu, var = _comb2(agg2, cnt3, r2, r3, Wl2, Wl3)
    return (mu, var)


# speedup vs baseline: 1.0109x; 1.0109x over previous
import functools

import jax
import jax.numpy as jnp
from jax import lax
from jax.experimental import pallas as pl
from jax.experimental.pallas import tpu as pltpu
from jax.experimental.pallas import tpu_sc as plsc

N_NODES = 10000
D = 128
N_EDGES = 320000

NC = 2
NS = 16
NW = NC * NS
CHUNK = 128
NROWS = N_EDGES // CHUNK
KCH = 80
LAST_KCH = NROWS - (NW - 1) * KCH
NROWS_PAD = NW * KCH
NP = 10240
RPT = NP // NS


def _make_sc_aggregate(with_cnt):
    mesh = plsc.VectorSubcoreMesh(core_axis_name="c", subcore_axis_name="s")

    out_type = [jax.ShapeDtypeStruct((NC, NP, D), jnp.float32)]
    scratch = [
        pltpu.VMEM((KCH // 2, CHUNK), jnp.int32),
        pltpu.VMEM((KCH // 2, CHUNK), jnp.int32),
        pltpu.VMEM((CHUNK, D), jnp.float32),
        pltpu.VMEM((CHUNK, D), jnp.float32),
        pltpu.VMEM_SHARED((NP, D), jnp.float32),
        pltpu.SemaphoreType.DMA,
        pltpu.SemaphoreType.DMA,
    ]
    if with_cnt:
        out_type.append(jax.ShapeDtypeStruct((NC, NP), jnp.float32))
        scratch += [
            pltpu.VMEM((CHUNK,), jnp.float32),
            pltpu.VMEM((RPT,), jnp.float32),
            pltpu.VMEM_SHARED((NP,), jnp.float32),
        ]

    @functools.partial(pl.kernel, out_type=out_type, mesh=mesh,
                       scratch_types=scratch)
    def agg_kernel(x_hbm, e_hbm, out_agg, *rest):
        if with_cnt:
            out_cnt, si, di, rows_a, rows_b, acc, sem_a, sem_b, \
                ones_v, zv, cacc = rest
        else:
            si, di, rows_a, rows_b, acc, sem_a, sem_b = rest
        cid = lax.axis_index("c")
        sid = lax.axis_index("s")
        wid = sid * NC + cid
        nch = jnp.where(wid == NW - 1, LAST_KCH, KCH)
        tb = sid * RPT

        zero16 = jnp.zeros((16,), jnp.float32)

        def fill_rows(i, _):
            rows_a[i // 8, pl.ds((i % 8) * 16, 16)] = zero16
            return _
        lax.fori_loop(0, CHUNK * 8, fill_rows, None)

        if with_cnt:
            def fill_ones(i, _):
                ones_v[pl.ds(i * 16, 16)] = jnp.full((16,), 1.0, jnp.float32)
                return _
            lax.fori_loop(0, CHUNK // 16, fill_ones, None)

            def fill_z(i, _):
                zv[pl.ds(i * 16, 16)] = zero16
                return _
            lax.fori_loop(0, RPT // 16, fill_z, None)

        zcp = [pltpu.async_copy(rows_a, acc.at[pl.ds(tb + b * CHUNK, CHUNK)],
                                sem_a) for b in range(RPT // CHUNK)]
        if with_cnt:
            zcp.append(pltpu.async_copy(zv, cacc.at[pl.ds(tb, RPT)], sem_a))
        for c in zcp:
            c.wait()
        plsc.subcore_barrier()

        half = KCH // 2
        last = half - 1

        def half_body(h, _):
            base = wid * KCH + h * half
            s1 = pltpu.async_copy(e_hbm.at[0, pl.ds(base, half)], si, sem_a)
            s2 = pltpu.async_copy(e_hbm.at[1, pl.ds(base, half)], di, sem_b)
            s1.wait()
            s2.wait()
            pltpu.async_copy(x_hbm.at[si.at[0]], rows_a, sem_a).wait()
            npairs = jnp.clip(nch - h * half, 0, half) // 2

            def chunk_body(i, __):
                j = 2 * i
                nxt_b = pltpu.async_copy(x_hbm.at[si.at[j + 1]], rows_b, sem_b)
                pltpu.sync_copy(rows_a, acc.at[di.at[j]], add=True)
                if with_cnt:
                    pltpu.sync_copy(ones_v, cacc.at[di.at[j]], add=True)
                nxt_b.wait()
                nxt_a = pltpu.async_copy(
                    x_hbm.at[si.at[jnp.minimum(j + 2, last)]], rows_a, sem_a)
                pltpu.sync_copy(rows_b, acc.at[di.at[j + 1]], add=True)
                if with_cnt:
                    pltpu.sync_copy(ones_v, cacc.at[di.at[j + 1]], add=True)
                nxt_a.wait()
                return __
            lax.fori_loop(0, npairs, chunk_body, None)
            return _
        lax.fori_loop(0, 2, half_body, None)

        plsc.subcore_barrier()

        for b in range(RPT // CHUNK):
            pltpu.sync_copy(acc.at[pl.ds(tb + b * CHUNK, CHUNK)],
                            out_agg.at[cid, pl.ds(tb + b * CHUNK, CHUNK)])
        if with_cnt:
            pltpu.sync_copy(cacc.at[pl.ds(tb, RPT)],
                            out_cnt.at[cid, pl.ds(tb, RPT)])

    return agg_kernel


_sc_agg_cnt = _make_sc_aggregate(True)


_BLK = 2000
_GRID = N_NODES // _BLK


def _dot_t(a, w):
    return lax.dot_general(a, w, (((1,), (1,)), ((), ())),
                           preferred_element_type=jnp.float32)


def _root1_body(x_ref, wr_ref, b_ref, o_ref):
    o_ref[...] = _dot_t(x_ref[...], wr_ref[...]) + b_ref[...]


def _root2_body(h_ref, wr2_ref, b2_ref, wr3_ref, b3_ref, o2_ref, o3_ref):
    h = h_ref[...]
    o2_ref[...] = _dot_t(h, wr2_ref[...]) + b2_ref[...]
    o3_ref[...] = _dot_t(h, wr3_ref[...]) + b3_ref[...]


def _comb1_body(a_ref, c_ref, r_ref, wl_ref, o_ref):
    a = a_ref[0] + a_ref[1]
    c = c_ref[0] + c_ref[1]
    mean = a / jnp.maximum(c, 1.0)
    o_ref[...] = jnp.maximum(_dot_t(mean, wl_ref[...]) + r_ref[...], 0.0)


def _comb2_body(a_ref, c_ref, r2_ref, r3_ref, wl2_ref, wl3_ref,
                mu_ref, var_ref):
    a = a_ref[0] + a_ref[1]
    c = c_ref[0] + c_ref[1]
    mean = a / jnp.maximum(c, 1.0)
    mu_ref[...] = _dot_t(mean, wl2_ref[...]) + r2_ref[...]
    var_ref[...] = _dot_t(mean, wl3_ref[...]) + r3_ref[...]


def _agg_spec():
    return pl.BlockSpec((NC, _BLK, D), lambda i: (0, i, 0))


def _cnt_spec():
    return pl.BlockSpec((NC, _BLK, 1), lambda i: (0, i, 0))


def _row_spec():
    return pl.BlockSpec((_BLK, D), lambda i: (i, 0))


def _w_spec():
    return pl.BlockSpec((D, D), lambda i: (0, 0))


def _b_spec():
    return pl.BlockSpec((1, D), lambda i: (0, 0))


_ROW_SDS = jax.ShapeDtypeStruct((N_NODES, D), jnp.float32)


def _root1(x, Wr, bl):
    return pl.pallas_call(
        _root1_body,
        grid=(_GRID,),
        in_specs=[_row_spec(), _w_spec(), _b_spec()],
        out_specs=_row_spec(),
        out_shape=_ROW_SDS,
    )(x, Wr, bl.reshape(1, D))


def _root2(h, Wr2, bl2, Wr3, bl3):
    return pl.pallas_call(
        _root2_body,
        grid=(_GRID,),
        in_specs=[_row_spec(), _w_spec(), _b_spec(), _w_spec(), _b_spec()],
        out_specs=[_row_spec(), _row_spec()],
        out_shape=[_ROW_SDS, _ROW_SDS],
    )(h, Wr2, bl2.reshape(1, D), Wr3, bl3.reshape(1, D))


def _comb1(agg, cnt3, r1, Wl):
    return pl.pallas_call(
        _comb1_body,
        grid=(_GRID,),
        in_specs=[_agg_spec(), _cnt_spec(), _row_spec(), _w_spec()],
        out_specs=_row_spec(),
        out_shape=_ROW_SDS,
    )(agg, cnt3, r1, Wl)


def _comb2(agg, cnt3, r2, r3, Wl2, Wl3):
    return pl.pallas_call(
        _comb2_body,
        grid=(_GRID,),
        in_specs=[_agg_spec(), _cnt_spec(), _row_spec(), _row_spec(),
                  _w_spec(), _w_spec()],
        out_specs=[_row_spec(), _row_spec()],
        out_shape=[_ROW_SDS, _ROW_SDS],
    )(agg, cnt3, r2, r3, Wl2, Wl3)


def kernel(x, edge_index, edge_weight, Wl1, bl1, Wr1, Wl2, bl2, Wr2,
           Wl3, bl3, Wr3):
    e3 = jnp.pad(edge_index.astype(jnp.int32).reshape(2, NROWS, CHUNK),
                 ((0, 0), (0, NROWS_PAD - NROWS), (0, 0)))

    r1 = _root1(x, Wr1, bl1)
    agg1, cnt = _sc_agg_cnt(x, e3)
    cnt3 = cnt[:, :, None]
    h1 = _comb1(agg1, cnt3, r1, Wl1)
    r2, r3 = _root2(h1, Wr2, bl2, Wr3, bl3)
    agg2, _ = _sc_agg_cnt(h1, e3)
    mu, var = _comb2(agg2, cnt3, r2, r3, Wl2, Wl3)
    return (mu, var)


# speedup vs baseline: 1.0177x; 1.0067x over previous
import functools

import jax
import jax.numpy as jnp
from jax import lax
from jax.experimental import pallas as pl
from jax.experimental.pallas import tpu as pltpu
from jax.experimental.pallas import tpu_sc as plsc

N_NODES = 10000
D = 128
N_EDGES = 320000

NC = 2
NS = 16
NW = NC * NS
CHUNK = 128
NROWS = N_EDGES // CHUNK
KCH = 80
LAST_KCH = NROWS - (NW - 1) * KCH
NROWS_PAD = NW * KCH
NP = 10240
RPT = NP // NS


def _make_sc_aggregate(with_cnt):
    mesh = plsc.VectorSubcoreMesh(core_axis_name="c", subcore_axis_name="s")

    out_type = [jax.ShapeDtypeStruct((NC, NP, D), jnp.float32)]
    scratch = [
        pltpu.VMEM((KCH // 2, CHUNK), jnp.int32),
        pltpu.VMEM((KCH // 2, CHUNK), jnp.int32),
        pltpu.VMEM((CHUNK, D), jnp.float32),
        pltpu.VMEM((CHUNK, D), jnp.float32),
        pltpu.VMEM_SHARED((NP, D), jnp.float32),
        pltpu.SemaphoreType.DMA,
        pltpu.SemaphoreType.DMA,
    ]
    if with_cnt:
        out_type.append(jax.ShapeDtypeStruct((NC, NP), jnp.float32))
        scratch += [
            pltpu.VMEM((CHUNK,), jnp.float32),
            pltpu.VMEM((RPT,), jnp.float32),
            pltpu.VMEM_SHARED((NP,), jnp.float32),
        ]

    @functools.partial(pl.kernel, out_type=out_type, mesh=mesh,
                       scratch_types=scratch)
    def agg_kernel(x_hbm, e_hbm, out_agg, *rest):
        if with_cnt:
            out_cnt, si, di, rows_a, rows_b, acc, sem_a, sem_b, \
                ones_v, zv, cacc = rest
        else:
            si, di, rows_a, rows_b, acc, sem_a, sem_b = rest
        cid = lax.axis_index("c")
        sid = lax.axis_index("s")
        wid = sid * NC + cid
        nch = jnp.where(wid == NW - 1, LAST_KCH, KCH)
        tb = sid * RPT

        zero16 = jnp.zeros((16,), jnp.float32)

        def fill_rows(i, _):
            rows_a[i // 8, pl.ds((i % 8) * 16, 16)] = zero16
            return _
        lax.fori_loop(0, CHUNK * 8, fill_rows, None)

        if with_cnt:
            def fill_ones(i, _):
                ones_v[pl.ds(i * 16, 16)] = jnp.full((16,), 1.0, jnp.float32)
                return _
            lax.fori_loop(0, CHUNK // 16, fill_ones, None)

            def fill_z(i, _):
                zv[pl.ds(i * 16, 16)] = zero16
                return _
            lax.fori_loop(0, RPT // 16, fill_z, None)

        zcp = [pltpu.async_copy(rows_a, acc.at[pl.ds(tb + b * CHUNK, CHUNK)],
                                sem_a) for b in range(RPT // CHUNK)]
        if with_cnt:
            zcp.append(pltpu.async_copy(zv, cacc.at[pl.ds(tb, RPT)], sem_a))
        for c in zcp:
            c.wait()
        plsc.subcore_barrier()

        half = KCH // 2
        last = half - 1

        def half_body(h, _):
            base = wid * KCH + h * half
            s1 = pltpu.async_copy(e_hbm.at[0, pl.ds(base, half)], si, sem_a)
            s2 = pltpu.async_copy(e_hbm.at[1, pl.ds(base, half)], di, sem_b)
            s1.wait()
            s2.wait()
            pltpu.async_copy(x_hbm.at[si.at[0]], rows_a, sem_a).wait()
            npairs = jnp.clip(nch - h * half, 0, half) // 2

            def chunk_body(i, __):
                j = 2 * i
                nxt_b = pltpu.async_copy(x_hbm.at[si.at[j + 1]], rows_b, sem_b)
                pltpu.sync_copy(rows_a, acc.at[di.at[j]], add=True)
                if with_cnt:
                    pltpu.sync_copy(ones_v, cacc.at[di.at[j]], add=True)
                nxt_b.wait()
                nxt_a = pltpu.async_copy(
                    x_hbm.at[si.at[jnp.minimum(j + 2, last)]], rows_a, sem_a)
                pltpu.sync_copy(rows_b, acc.at[di.at[j + 1]], add=True)
                if with_cnt:
                    pltpu.sync_copy(ones_v, cacc.at[di.at[j + 1]], add=True)
                nxt_a.wait()
                return __
            lax.fori_loop(0, npairs, chunk_body, None)
            return _
        lax.fori_loop(0, 2, half_body, None)

        plsc.subcore_barrier()

        for b in range(RPT // CHUNK):
            pltpu.sync_copy(acc.at[pl.ds(tb + b * CHUNK, CHUNK)],
                            out_agg.at[cid, pl.ds(tb + b * CHUNK, CHUNK)])
        if with_cnt:
            pltpu.sync_copy(cacc.at[pl.ds(tb, RPT)],
                            out_cnt.at[cid, pl.ds(tb, RPT)])

    return agg_kernel


_sc_agg_cache = []


def _sc_agg_cnt(x, e3):
    if not _sc_agg_cache:
        _sc_agg_cache.append(_make_sc_aggregate(True))
    return _sc_agg_cache[0](x, e3)


_BLK = 2000
_GRID = N_NODES // _BLK


def _dot_t(a, w):
    return lax.dot_general(a, w, (((1,), (1,)), ((), ())),
                           preferred_element_type=jnp.float32)


def _root1_body(x_ref, wr_ref, b_ref, o_ref):
    o_ref[...] = _dot_t(x_ref[...], wr_ref[...]) + b_ref[...]


def _root2_body(h_ref, wr2_ref, b2_ref, wr3_ref, b3_ref, o2_ref, o3_ref):
    h = h_ref[...]
    o2_ref[...] = _dot_t(h, wr2_ref[...]) + b2_ref[...]
    o3_ref[...] = _dot_t(h, wr3_ref[...]) + b3_ref[...]


def _comb1_body(a_ref, c_ref, r_ref, wl_ref, o_ref):
    a = a_ref[0] + a_ref[1]
    c = c_ref[0] + c_ref[1]
    mean = a / jnp.maximum(c, 1.0)
    o_ref[...] = jnp.maximum(_dot_t(mean, wl_ref[...]) + r_ref[...], 0.0)


def _comb2_body(a_ref, c_ref, r2_ref, r3_ref, wl2_ref, wl3_ref,
                mu_ref, var_ref):
    a = a_ref[0] + a_ref[1]
    c = c_ref[0] + c_ref[1]
    mean = a / jnp.maximum(c, 1.0)
    mu_ref[...] = _dot_t(mean, wl2_ref[...]) + r2_ref[...]
    var_ref[...] = _dot_t(mean, wl3_ref[...]) + r3_ref[...]


def _agg_spec():
    return pl.BlockSpec((NC, _BLK, D), lambda i: (0, i, 0))


def _cnt_spec():
    return pl.BlockSpec((NC, _BLK, 1), lambda i: (0, i, 0))


def _row_spec():
    return pl.BlockSpec((_BLK, D), lambda i: (i, 0))


def _w_spec():
    return pl.BlockSpec((D, D), lambda i: (0, 0))


def _b_spec():
    return pl.BlockSpec((1, D), lambda i: (0, 0))


_ROW_SDS = jax.ShapeDtypeStruct((N_NODES, D), jnp.float32)


def _root1(x, Wr, bl):
    return pl.pallas_call(
        _root1_body,
        grid=(_GRID,),
        in_specs=[_row_spec(), _w_spec(), _b_spec()],
        out_specs=_row_spec(),
        out_shape=_ROW_SDS,
    )(x, Wr, bl.reshape(1, D))


def _root2(h, Wr2, bl2, Wr3, bl3):
    return pl.pallas_call(
        _root2_body,
        grid=(_GRID,),
        in_specs=[_row_spec(), _w_spec(), _b_spec(), _w_spec(), _b_spec()],
        out_specs=[_row_spec(), _row_spec()],
        out_shape=[_ROW_SDS, _ROW_SDS],
    )(h, Wr2, bl2.reshape(1, D), Wr3, bl3.reshape(1, D))


def _comb1(agg, cnt3, r1, Wl):
    return pl.pallas_call(
        _comb1_body,
        grid=(_GRID,),
        in_specs=[_agg_spec(), _cnt_spec(), _row_spec(), _w_spec()],
        out_specs=_row_spec(),
        out_shape=_ROW_SDS,
    )(agg, cnt3, r1, Wl)


def _comb2(agg, cnt3, r2, r3, Wl2, Wl3):
    return pl.pallas_call(
        _comb2_body,
        grid=(_GRID,),
        in_specs=[_agg_spec(), _cnt_spec(), _row_spec(), _row_spec(),
                  _w_spec(), _w_spec()],
        out_specs=[_row_spec(), _row_spec()],
        out_shape=[_ROW_SDS, _ROW_SDS],
    )(agg, cnt3, r2, r3, Wl2, Wl3)


def kernel(x, edge_index, edge_weight, Wl1, bl1, Wr1, Wl2, bl2, Wr2,
           Wl3, bl3, Wr3):
    e3 = jnp.pad(edge_index.astype(jnp.int32).reshape(2, NROWS, CHUNK),
                 ((0, 0), (0, NROWS_PAD - NROWS), (0, 0)))

    r1 = _root1(x, Wr1, bl1)
    agg1, cnt = _sc_agg_cnt(x, e3)
    cnt3 = cnt[:, :, None]
    h1 = _comb1(agg1, cnt3, r1, Wl1)
    r2, r3 = _root2(h1, Wr2, bl2, Wr3, bl3)
    agg2, _ = _sc_agg_cnt(h1, e3)
    mu, var = _comb2(agg2, cnt3, r2, r3, Wl2, Wl3)
    return (mu, var)
